# baseline (speedup 1.0000x reference)
import functools

import jax
import jax.numpy as jnp
from jax import lax
from jax.experimental import pallas as pl
from jax.experimental.pallas import tpu as pltpu
from jax.experimental.pallas import tpu_sc as plsc

B = 16384
NC, NS, L = 2, 16, 16
NW = NC * NS
BPW = B // NW
CHUNK = 128
NCHUNK = BPW // CHUNK

VOC = [1000, 8, 30, 40, 8, 4, 20, 20, 4, 10000, 7]
DIM = [100, 1, 3, 4, 1, 1, 2, 2, 1, 1000, 1]
COL = [0, 100, 101, 104, 108, 109, 110, 112, 114, 115, 1115]
HOUR_COL = 1116

SMALL = [0, 1, 2, 3, 4, 5, 6, 7, 8, 10]
TINY = [1, 2, 3, 4, 5, 6, 7, 8, 10]
SOFF = {0: 0, 1: 1000, 2: 1008, 3: 1040, 4: 1080, 5: 1088,
        6: 1096, 7: 1120, 8: 1144, 10: 1152}
PSMALL_ROWS = 1160
K_BLK = 200
K_GRID = DIM[9] // K_BLK


def _proj_body(*refs):
    t9t, w9t, t0t, w0t, w1 = refs[:5]
    tiny = refs[5:5 + len(TINY)]
    pbig, psmall, ptacc = refs[-3], refs[-2], refs[-1]
    i = pl.program_id(0)
    part = lax.dot_general(w9t[...], t9t[...], (((0,), (0,)), ((), ())),
                           preferred_element_type=jnp.float32)

    @pl.when(i == 0)
    def _():
        ptacc[...] = part

    @pl.when(i != 0)
    def _():
        ptacc[...] += part

    @pl.when(i == K_GRID - 1)
    def _():
        pbig[...] = jnp.pad(ptacc[...].T, ((0, 0), (0, L - 10)))

    @pl.when(i == 0)
    def _():
        p0 = lax.dot_general(w0t[...], t0t[...], (((0,), (0,)), ((), ())),
                             preferred_element_type=jnp.float32)
        psmall[SOFF[0]:SOFF[0] + VOC[0], :] = jnp.pad(p0.T, ((0, 0), (0, L - 10)))
        for k, t in enumerate(TINY):
            wsl = w1[:, COL[t]:COL[t] + DIM[t]]
            r = lax.dot_general(tiny[k][...], wsl, (((0,), (1,)), ((), ())),
                                preferred_element_type=jnp.float32)
            psmall[SOFF[t]:SOFF[t] + VOC[t], :] = jnp.pad(r, ((0, 0), (0, L - 10)))


def _project(t9t, w9t, t0t, w0t, W1, tinies):
    const = lambda s: pl.BlockSpec(s, lambda i: (0,) * len(s))
    in_specs = [pl.BlockSpec((K_BLK, VOC[9]), lambda i: (i, 0)),
                pl.BlockSpec((K_BLK, 10), lambda i: (i, 0)),
                const(t0t.shape), const(w0t.shape), const(W1.shape)]
    args = [t9t, w9t, t0t, w0t, W1]
    for tab in tinies:
        in_specs.append(const(tab.shape))
        args.append(tab)
    return pl.pallas_call(
        _proj_body,
        grid=(K_GRID,),
        in_specs=in_specs,
        out_specs=[pl.BlockSpec((VOC[9], L), lambda i: (0, 0)),
                   pl.BlockSpec((PSMALL_ROWS, L), lambda i: (0, 0))],
        out_shape=[jax.ShapeDtypeStruct((VOC[9], L), jnp.float32),
                   jax.ShapeDtypeStruct((PSMALL_ROWS, L), jnp.float32)],
        scratch_shapes=[pltpu.VMEM((10, VOC[9]), jnp.float32)],
    )(*args)


def _fused_body(psmall_hbm, pbig_hbm, idx9_hbm, *rest):
    idx_hbm = rest[:len(SMALL)]
    hour_hbm, wpack_hbm, y_hbm = rest[len(SMALL):len(SMALL) + 3]
    (psv, t9rows, idxv, idx9v, hourv, wv, outv,
     semA, s0, s1, s2, s3) = rest[len(SMALL) + 3:]
    sem9 = [s0, s1, s2, s3]

    sid = lax.axis_index("s")
    wid = sid * NC + lax.axis_index("c")
    base = wid * BPW

    pltpu.sync_copy(idx9_hbm.at[pl.ds(wid * NCHUNK, NCHUNK)], idx9v)
    cps9 = [pltpu.async_copy(pbig_hbm.at[idx9v.at[c]],
                             t9rows.at[pl.ds(c * CHUNK, CHUNK)], sem9[c])
            for c in range(NCHUNK)]

    cps = [pltpu.async_copy(psmall_hbm, psv, semA),
           pltpu.async_copy(hour_hbm.at[pl.ds(base, BPW)], hourv, semA),
           pltpu.async_copy(wpack_hbm, wv, semA)]
    cps += [pltpu.async_copy(idx_hbm[k].at[pl.ds(base, BPW)], idxv.at[k], semA)
            for k in range(len(SMALL))]
    for cp in cps:
        cp.wait()

    iota = lax.iota(jnp.int32, L)
    w1hvec = wv[pl.ds(0, L)]
    b1vec = wv[pl.ds(16, L)]
    mixvec = wv[pl.ds(32, L)]
    w2vecs = [wv[pl.ds(48 + L * i, L)] for i in range(4)]

    def w2el(j, k):
        f = j * 10 + k
        return w2vecs[f // L][f % L]

    cj = [jnp.full((L,), j, jnp.int32) for j in range(10)]

    def group_body(g):
        rows = iota + g * L
        hvec = hourv[pl.ds(g * L, L)]
        ids = [idxv[k, pl.ds(g * L, L)] + SOFF[t]
               for k, t in enumerate(SMALL)]
        xs = []
        for j in range(10):
            gs = [plsc.load_gather(t9rows, [rows, cj[j]])]
            gs += [plsc.load_gather(psv, [ids[k], cj[j]])
                   for k in range(len(SMALL))]
            while len(gs) > 1:
                gs = [gs[i] + gs[i + 1] for i in range(0, len(gs) - 1, 2)] \
                    + ([gs[-1]] if len(gs) % 2 else [])
            xs.append(jnp.maximum(gs[0] + hvec * w1hvec[j] + b1vec[j], 0.0))
        h2 = []
        for j in range(5):
            t = xs[0] * w2el(j, 0)
            for k in range(1, 10):
                t = t + xs[k] * w2el(j, k)
            h2.append(jnp.maximum(t + mixvec[j], 0.0))
        o = h2[0] * mixvec[5]
        for j in range(1, 5):
            o = o + h2[j] * mixvec[5 + j]
        o = o + mixvec[10]
        outv[pl.ds(g * L, L)] = 1.0 / (1.0 + jnp.exp(-o))

    GPC = CHUNK // L
    for c in range(NCHUNK):
        cps9[c].wait()
        plsc.parallel_loop(c * GPC, (c + 1) * GPC, 1, unroll=1)(group_body)

    pltpu.sync_copy(outv, y_hbm.at[pl.ds(base, BPW)])


def _fused_sc(psmall, pbig, idx9r, idxs_small, hour, wpack):
    mesh = plsc.VectorSubcoreMesh(core_axis_name="c", subcore_axis_name="s")
    return pl.kernel(
        _fused_body,
        out_type=jax.ShapeDtypeStruct((B,), jnp.float32),
        mesh=mesh,
        compiler_params=pltpu.CompilerParams(use_tc_tiling_on_sc=False,
                                             needs_layout_passes=False,
                                             skip_device_barrier=True),
        name="fused_gather_mlp",
        scratch_types=[
            pltpu.VMEM((PSMALL_ROWS, L), jnp.float32),
            pltpu.VMEM((BPW, L), jnp.float32),
            pltpu.VMEM((len(SMALL), BPW), jnp.int32),
            pltpu.VMEM((NCHUNK, CHUNK), jnp.int32),
            pltpu.VMEM((BPW,), jnp.float32),
            pltpu.VMEM((112,), jnp.float32),
            pltpu.VMEM((BPW,), jnp.float32),
            pltpu.SemaphoreType.DMA,
            pltpu.SemaphoreType.DMA,
            pltpu.SemaphoreType.DMA,
            pltpu.SemaphoreType.DMA,
            pltpu.SemaphoreType.DMA,
        ],
    )(psmall, pbig, idx9r, *idxs_small, hour, wpack)


def kernel(idx_0, idx_1, idx_2, idx_3, idx_4, idx_5, idx_6, idx_7, idx_8,
           idx_9, idx_10, hour,
           table_0, table_1, table_2, table_3, table_4, table_5, table_6,
           table_7, table_8, table_9, table_10,
           W1, b1, W2, b2, W3, b3):
    idxs = [idx_0, idx_1, idx_2, idx_3, idx_4, idx_5, idx_6, idx_7, idx_8,
            idx_9, idx_10]
    tables = [table_0, table_1, table_2, table_3, table_4, table_5, table_6,
              table_7, table_8, table_9, table_10]

    W1T = W1.T
    w9t = W1T[COL[9]:COL[9] + DIM[9]]
    w0t = W1T[COL[0]:COL[0] + DIM[0]]
    w1h = W1T[HOUR_COL]

    pbig, psmall = _project(table_9.T, w9t, table_0.T, w0t, W1,
                            [tables[t].T for t in TINY])

    z6 = jnp.zeros((6,), jnp.float32)
    wpack = jnp.concatenate([
        w1h, z6, b1, z6, b2, W3.reshape(5), b3, jnp.zeros((5,), jnp.float32),
        W2.reshape(50), jnp.zeros((14,), jnp.float32)])

    y = _fused_sc(psmall, pbig, idxs[9].reshape(B // CHUNK, CHUNK),
                  [idxs[t] for t in SMALL], hour, wpack)
    return y.reshape(B, 1)


# speedup vs baseline: 1.1013x; 1.1013x over previous
import functools

import jax
import jax.numpy as jnp
from jax import lax
from jax.experimental import pallas as pl
from jax.experimental.pallas import tpu as pltpu
from jax.experimental.pallas import tpu_sc as plsc

B = 16384
NC, NS, L = 2, 16, 16
NW = NC * NS
BPW = B // NW
CHUNK = 128
NCHUNK = BPW // CHUNK

VOC = [1000, 8, 30, 40, 8, 4, 20, 20, 4, 10000, 7]
DIM = [100, 1, 3, 4, 1, 1, 2, 2, 1, 1000, 1]
COL = [0, 100, 101, 104, 108, 109, 110, 112, 114, 115, 1115]
HOUR_COL = 1116

SMALL = [0, 1, 2, 3, 4, 5, 6, 7, 8, 10]
TINY = [1, 2, 3, 4, 5, 6, 7, 8, 10]
SOFF = {0: 0, 1: 1000, 2: 1008, 3: 1040, 4: 1080, 5: 1088,
        6: 1096, 7: 1120, 8: 1144, 10: 1152}
PSMALL_ROWS = 1160
K_BLK = 200
K_GRID = DIM[9] // K_BLK


def _proj_body(*refs):
    t9t, w9t, t0t, w0t, w1 = refs[:5]
    tiny = refs[5:5 + len(TINY)]
    pbig, psmall, ptacc = refs[-3], refs[-2], refs[-1]
    i = pl.program_id(0)
    part = lax.dot_general(w9t[...], t9t[...], (((0,), (0,)), ((), ())),
                           preferred_element_type=jnp.float32)

    @pl.when(i == 0)
    def _():
        ptacc[...] = part

    @pl.when(i != 0)
    def _():
        ptacc[...] += part

    @pl.when(i == K_GRID - 1)
    def _():
        pbig[...] = jnp.pad(ptacc[...].T, ((0, 0), (0, L - 10)))

    @pl.when(i == 0)
    def _():
        p0 = lax.dot_general(w0t[...], t0t[...], (((0,), (0,)), ((), ())),
                             preferred_element_type=jnp.float32)
        psmall[SOFF[0]:SOFF[0] + VOC[0], :] = jnp.pad(p0.T, ((0, 0), (0, L - 10)))
        for k, t in enumerate(TINY):
            wsl = w1[:, COL[t]:COL[t] + DIM[t]]
            r = lax.dot_general(tiny[k][...], wsl, (((0,), (1,)), ((), ())),
                                preferred_element_type=jnp.float32)
            psmall[SOFF[t]:SOFF[t] + VOC[t], :] = jnp.pad(r, ((0, 0), (0, L - 10)))


def _project(t9t, w9t, t0t, w0t, W1, tinies):
    const = lambda s: pl.BlockSpec(s, lambda i: (0,) * len(s))
    in_specs = [pl.BlockSpec((K_BLK, VOC[9]), lambda i: (i, 0)),
                pl.BlockSpec((K_BLK, 10), lambda i: (i, 0)),
                const(t0t.shape), const(w0t.shape), const(W1.shape)]
    args = [t9t, w9t, t0t, w0t, W1]
    for tab in tinies:
        in_specs.append(const(tab.shape))
        args.append(tab)
    return pl.pallas_call(
        _proj_body,
        grid=(K_GRID,),
        in_specs=in_specs,
        out_specs=[pl.BlockSpec((VOC[9], L), lambda i: (0, 0)),
                   pl.BlockSpec((PSMALL_ROWS, L), lambda i: (0, 0))],
        out_shape=[jax.ShapeDtypeStruct((VOC[9], L), jnp.float32),
                   jax.ShapeDtypeStruct((PSMALL_ROWS, L), jnp.float32)],
        scratch_shapes=[pltpu.VMEM((10, VOC[9]), jnp.float32)],
    )(*args)


def _fused_body(psmall_hbm, pbig_hbm, idx9_hbm, *rest):
    idx_hbm = rest[:len(SMALL)]
    hour_hbm, wpack_hbm, y_hbm = rest[len(SMALL):len(SMALL) + 3]
    (psv, t9rows, idxv, idx9v, hourv, wv, outv,
     semA, s0, s1, s2, s3) = rest[len(SMALL) + 3:]
    sem9 = [s0, s1, s2, s3]

    sid = lax.axis_index("s")
    wid = sid * NC + lax.axis_index("c")
    base = wid * BPW

    pltpu.sync_copy(idx9_hbm.at[pl.ds(wid * NCHUNK, NCHUNK)], idx9v)
    cps9 = [pltpu.async_copy(pbig_hbm.at[idx9v.at[c]],
                             t9rows.at[pl.ds(c * CHUNK, CHUNK)], sem9[c])
            for c in range(NCHUNK)]

    cps = [pltpu.async_copy(psmall_hbm, psv, semA),
           pltpu.async_copy(hour_hbm.at[pl.ds(base, BPW)], hourv, semA),
           pltpu.async_copy(wpack_hbm, wv, semA)]
    cps += [pltpu.async_copy(idx_hbm[k].at[pl.ds(base, BPW)], idxv.at[k], semA)
            for k in range(len(SMALL))]
    for cp in cps:
        cp.wait()

    iota = lax.iota(jnp.int32, L)
    w1hvec = wv[pl.ds(0, L)]
    b1vec = wv[pl.ds(16, L)]
    mixvec = wv[pl.ds(32, L)]
    w2vecs = [wv[pl.ds(48 + L * i, L)] for i in range(4)]

    def w2el(j, k):
        f = j * 10 + k
        return w2vecs[f // L][f % L]

    cj = [jnp.full((L,), j, jnp.int32) for j in range(10)]

    def group_body(g):
        rows = iota + g * L
        hvec = hourv[pl.ds(g * L, L)]
        ids = [idxv[k, pl.ds(g * L, L)] + SOFF[t]
               for k, t in enumerate(SMALL)]
        xs = []
        for j in range(10):
            v = plsc.load_gather(t9rows, [rows, cj[j]])
            for k in range(len(SMALL)):
                v = v + plsc.load_gather(psv, [ids[k], cj[j]])
            xs.append(jnp.maximum(v + hvec * w1hvec[j] + b1vec[j], 0.0))
        h2 = []
        for j in range(5):
            t = xs[0] * w2el(j, 0)
            for k in range(1, 10):
                t = t + xs[k] * w2el(j, k)
            h2.append(jnp.maximum(t + mixvec[j], 0.0))
        o = h2[0] * mixvec[5]
        for j in range(1, 5):
            o = o + h2[j] * mixvec[5 + j]
        o = o + mixvec[10]
        outv[pl.ds(g * L, L)] = 1.0 / (1.0 + jnp.exp(-o))
        return 0

    GPC = CHUNK // L
    for c in range(NCHUNK):
        cps9[c].wait()
        lax.fori_loop(c * GPC, (c + 1) * GPC, lambda g, car: group_body(g) or car, 0)

    pltpu.sync_copy(outv, y_hbm.at[pl.ds(base, BPW)])


def _fused_sc(psmall, pbig, idx9r, idxs_small, hour, wpack):
    mesh = plsc.VectorSubcoreMesh(core_axis_name="c", subcore_axis_name="s")
    return pl.kernel(
        _fused_body,
        out_type=jax.ShapeDtypeStruct((B,), jnp.float32),
        mesh=mesh,
        compiler_params=pltpu.CompilerParams(use_tc_tiling_on_sc=False,
                                             needs_layout_passes=False,
                                             skip_device_barrier=True),
        name="fused_gather_mlp",
        scratch_types=[
            pltpu.VMEM((PSMALL_ROWS, L), jnp.float32),
            pltpu.VMEM((BPW, L), jnp.float32),
            pltpu.VMEM((len(SMALL), BPW), jnp.int32),
            pltpu.VMEM((NCHUNK, CHUNK), jnp.int32),
            pltpu.VMEM((BPW,), jnp.float32),
            pltpu.VMEM((112,), jnp.float32),
            pltpu.VMEM((BPW,), jnp.float32),
            pltpu.SemaphoreType.DMA,
            pltpu.SemaphoreType.DMA,
            pltpu.SemaphoreType.DMA,
            pltpu.SemaphoreType.DMA,
            pltpu.SemaphoreType.DMA,
        ],
    )(psmall, pbig, idx9r, *idxs_small, hour, wpack)


def kernel(idx_0, idx_1, idx_2, idx_3, idx_4, idx_5, idx_6, idx_7, idx_8,
           idx_9, idx_10, hour,
           table_0, table_1, table_2, table_3, table_4, table_5, table_6,
           table_7, table_8, table_9, table_10,
           W1, b1, W2, b2, W3, b3):
    idxs = [idx_0, idx_1, idx_2, idx_3, idx_4, idx_5, idx_6, idx_7, idx_8,
            idx_9, idx_10]
    tables = [table_0, table_1, table_2, table_3, table_4, table_5, table_6,
              table_7, table_8, table_9, table_10]

    W1T = W1.T
    w9t = W1T[COL[9]:COL[9] + DIM[9]]
    w0t = W1T[COL[0]:COL[0] + DIM[0]]
    w1h = W1T[HOUR_COL]

    pbig, psmall = _project(table_9.T, w9t, table_0.T, w0t, W1,
                            [tables[t].T for t in TINY])

    z6 = jnp.zeros((6,), jnp.float32)
    wpack = jnp.concatenate([
        w1h, z6, b1, z6, b2, W3.reshape(5), b3, jnp.zeros((5,), jnp.float32),
        W2.reshape(50), jnp.zeros((14,), jnp.float32)])

    y = _fused_sc(psmall, pbig, idxs[9].reshape(B // CHUNK, CHUNK),
                  [idxs[t] for t in SMALL], hour, wpack)
    return y.reshape(B, 1)


# speedup vs baseline: 1.1358x; 1.0313x over previous
import functools

import jax
import jax.numpy as jnp
from jax import lax
from jax.experimental import pallas as pl
from jax.experimental.pallas import tpu as pltpu
from jax.experimental.pallas import tpu_sc as plsc

B = 16384
NC, NS, L = 2, 16, 16
NW = NC * NS
BPW = B // NW
CHUNK = 128
NCHUNK = BPW // CHUNK

VOC = [1000, 8, 30, 40, 8, 4, 20, 20, 4, 10000, 7]
DIM = [100, 1, 3, 4, 1, 1, 2, 2, 1, 1000, 1]
COL = [0, 100, 101, 104, 108, 109, 110, 112, 114, 115, 1115]
HOUR_COL = 1116

SMALL = [0, 1, 2, 3, 4, 5, 6, 7, 8, 10]
TINY = [1, 2, 3, 4, 5, 6, 7, 8, 10]
SOFF = {0: 0, 1: 1000, 2: 1008, 3: 1040, 4: 1080, 5: 1088,
        6: 1096, 7: 1120, 8: 1144, 10: 1152}
PSMALL_ROWS = 1160
WIDE = 128
K_BLK = 200
K_GRID = DIM[9] // K_BLK


def _proj_body(*refs):
    t9t, w9t, t0t, w0t, w1 = refs[:5]
    tiny = refs[5:5 + len(TINY)]
    pbig, psmall, ptacc = refs[-3], refs[-2], refs[-1]
    i = pl.program_id(0)
    part = lax.dot_general(w9t[...], t9t[...], (((0,), (0,)), ((), ())),
                           preferred_element_type=jnp.float32)

    @pl.when(i == 0)
    def _():
        ptacc[...] = part

    @pl.when(i != 0)
    def _():
        ptacc[...] += part

    @pl.when(i == K_GRID - 1)
    def _():
        pbig[...] = jnp.pad(ptacc[...].T, ((0, 0), (0, WIDE - 10)))

    @pl.when(i == 0)
    def _():
        p0 = lax.dot_general(w0t[...], t0t[...], (((0,), (0,)), ((), ())),
                             preferred_element_type=jnp.float32)
        psmall[SOFF[0]:SOFF[0] + VOC[0], :] = jnp.pad(p0.T, ((0, 0), (0, L - 10)))
        for k, t in enumerate(TINY):
            wsl = w1[:, COL[t]:COL[t] + DIM[t]]
            r = lax.dot_general(tiny[k][...], wsl, (((0,), (1,)), ((), ())),
                                preferred_element_type=jnp.float32)
            psmall[SOFF[t]:SOFF[t] + VOC[t], :] = jnp.pad(r, ((0, 0), (0, L - 10)))


def _project(t9t, w9t, t0t, w0t, W1, tinies):
    const = lambda s: pl.BlockSpec(s, lambda i: (0,) * len(s))
    in_specs = [pl.BlockSpec((K_BLK, VOC[9]), lambda i: (i, 0)),
                pl.BlockSpec((K_BLK, 10), lambda i: (i, 0)),
                const(t0t.shape), const(w0t.shape), const(W1.shape)]
    args = [t9t, w9t, t0t, w0t, W1]
    for tab in tinies:
        in_specs.append(const(tab.shape))
        args.append(tab)
    return pl.pallas_call(
        _proj_body,
        grid=(K_GRID,),
        in_specs=in_specs,
        out_specs=[pl.BlockSpec((VOC[9], WIDE), lambda i: (0, 0)),
                   pl.BlockSpec((PSMALL_ROWS, L), lambda i: (0, 0))],
        out_shape=[jax.ShapeDtypeStruct((VOC[9], WIDE), jnp.float32),
                   jax.ShapeDtypeStruct((PSMALL_ROWS, L), jnp.float32)],
        scratch_shapes=[pltpu.VMEM((10, VOC[9]), jnp.float32)],
    )(*args)


def _fused_body(psmall_hbm, pbig_hbm, idx9_hbm, *rest):
    idx_hbm = rest[:len(SMALL)]
    hour_hbm, wpack_hbm, y_hbm = rest[len(SMALL):len(SMALL) + 3]
    (psv, t9rows, idxv, idx9v, hourv, wv, outv,
     semA, s0, s1, s2, s3) = rest[len(SMALL) + 3:]
    sem9 = [s0, s1, s2, s3]

    sid = lax.axis_index("s")
    wid = sid * NC + lax.axis_index("c")
    base = wid * BPW

    pltpu.sync_copy(idx9_hbm.at[pl.ds(wid * NCHUNK, NCHUNK)], idx9v)
    cps9 = [pltpu.async_copy(pbig_hbm.at[idx9v.at[c]],
                             t9rows.at[pl.ds(c * CHUNK, CHUNK)], sem9[c])
            for c in range(NCHUNK)]

    cps = [pltpu.async_copy(psmall_hbm, psv, semA),
           pltpu.async_copy(hour_hbm.at[pl.ds(base, BPW)], hourv, semA),
           pltpu.async_copy(wpack_hbm, wv, semA)]
    cps += [pltpu.async_copy(idx_hbm[k].at[pl.ds(base, BPW)], idxv.at[k], semA)
            for k in range(len(SMALL))]
    for cp in cps:
        cp.wait()

    iota = lax.iota(jnp.int32, L)
    w1hvec = wv[pl.ds(0, L)]
    b1vec = wv[pl.ds(16, L)]
    mixvec = wv[pl.ds(32, L)]
    w2vecs = [wv[pl.ds(48 + L * i, L)] for i in range(4)]

    def w2el(j, k):
        f = j * 10 + k
        return w2vecs[f // L][f % L]

    cj = [jnp.full((L,), j, jnp.int32) for j in range(10)]

    def group_body(g):
        rows = iota + g * L
        hvec = hourv[pl.ds(g * L, L)]
        ids = [idxv[k, pl.ds(g * L, L)] + SOFF[t]
               for k, t in enumerate(SMALL)]
        xs = []
        for j in range(10):
            v = plsc.load_gather(t9rows, [rows, cj[j]])
            for k in range(len(SMALL)):
                v = v + plsc.load_gather(psv, [ids[k], cj[j]])
            xs.append(jnp.maximum(v + hvec * w1hvec[j] + b1vec[j], 0.0))
        h2 = []
        for j in range(5):
            t = xs[0] * w2el(j, 0)
            for k in range(1, 10):
                t = t + xs[k] * w2el(j, k)
            h2.append(jnp.maximum(t + mixvec[j], 0.0))
        o = h2[0] * mixvec[5]
        for j in range(1, 5):
            o = o + h2[j] * mixvec[5 + j]
        o = o + mixvec[10]
        outv[pl.ds(g * L, L)] = 1.0 / (1.0 + jnp.exp(-o))
        return 0

    GPC = CHUNK // L
    for c in range(NCHUNK):
        cps9[c].wait()
        lax.fori_loop(c * GPC, (c + 1) * GPC, lambda g, car: group_body(g) or car, 0)

    pltpu.sync_copy(outv, y_hbm.at[pl.ds(base, BPW)])


def _fused_sc(psmall, pbig, idx9r, idxs_small, hour, wpack):
    mesh = plsc.VectorSubcoreMesh(core_axis_name="c", subcore_axis_name="s")
    return pl.kernel(
        _fused_body,
        out_type=jax.ShapeDtypeStruct((B,), jnp.float32),
        mesh=mesh,
        compiler_params=pltpu.CompilerParams(use_tc_tiling_on_sc=False,
                                             needs_layout_passes=False,
                                             skip_device_barrier=True),
        name="fused_gather_mlp",
        scratch_types=[
            pltpu.VMEM((PSMALL_ROWS, L), jnp.float32),
            pltpu.VMEM((BPW, WIDE), jnp.float32),
            pltpu.VMEM((len(SMALL), BPW), jnp.int32),
            pltpu.VMEM((NCHUNK, CHUNK), jnp.int32),
            pltpu.VMEM((BPW,), jnp.float32),
            pltpu.VMEM((112,), jnp.float32),
            pltpu.VMEM((BPW,), jnp.float32),
            pltpu.SemaphoreType.DMA,
            pltpu.SemaphoreType.DMA,
            pltpu.SemaphoreType.DMA,
            pltpu.SemaphoreType.DMA,
            pltpu.SemaphoreType.DMA,
        ],
    )(psmall, pbig, idx9r, *idxs_small, hour, wpack)


def kernel(idx_0, idx_1, idx_2, idx_3, idx_4, idx_5, idx_6, idx_7, idx_8,
           idx_9, idx_10, hour,
           table_0, table_1, table_2, table_3, table_4, table_5, table_6,
           table_7, table_8, table_9, table_10,
           W1, b1, W2, b2, W3, b3):
    idxs = [idx_0, idx_1, idx_2, idx_3, idx_4, idx_5, idx_6, idx_7, idx_8,
            idx_9, idx_10]
    tables = [table_0, table_1, table_2, table_3, table_4, table_5, table_6,
              table_7, table_8, table_9, table_10]

    W1T = W1.T
    w9t = W1T[COL[9]:COL[9] + DIM[9]]
    w0t = W1T[COL[0]:COL[0] + DIM[0]]
    w1h = W1T[HOUR_COL]

    pbig, psmall = _project(table_9.T, w9t, table_0.T, w0t, W1,
                            [tables[t].T for t in TINY])

    z6 = jnp.zeros((6,), jnp.float32)
    wpack = jnp.concatenate([
        w1h, z6, b1, z6, b2, W3.reshape(5), b3, jnp.zeros((5,), jnp.float32),
        W2.reshape(50), jnp.zeros((14,), jnp.float32)])

    y = _fused_sc(psmall, pbig, idxs[9].reshape(B // CHUNK, CHUNK),
                  [idxs[t] for t in SMALL], hour, wpack)
    return y.reshape(B, 1)
